# speedup vs baseline: 3.0660x; 3.0660x over previous
import jax
import jax.numpy as jnp
from jax.experimental import pallas as pl
from jax.experimental.pallas import tpu as pltpu

B, D, M = 1024, 64, 100000
TEMP, LAMBDA2, MU = 0.05, 0.5, 1.0
TM = 512
NT = (M + TM - 1) // TM
INV_TEMP = 1.0 / TEMP


def _norm_rows(x):
    n = jnp.sqrt(jnp.sum(x * x, axis=1, keepdims=True))
    return x / jnp.maximum(n, 1e-12)


def _fold4(e):
    return (e[:, 0:128] + e[:, 128:256]) + (e[:, 256:384] + e[:, 384:512])


def _loss_kernel(x_ref, xu_ref, xd_ref, t_ref, tu_ref, td_ref, tgt_ref,
                 f0_ref, f1_ref, f2_ref, out_ref,
                 xn0, xn1, xn2, acc0, acc1, acc2, ta0, ta1, ta2):
    j = pl.program_id(0)

    @pl.when(j == 0)
    def _init():
        xn0[...] = _norm_rows(x_ref[...]) * INV_TEMP
        xn1[...] = _norm_rows(xu_ref[...]) * INV_TEMP
        xn2[...] = _norm_rows(xd_ref[...]) * INV_TEMP
        for a in (acc0, acc1, acc2, ta0, ta1, ta2):
            a[...] = jnp.zeros_like(a)

    col = jax.lax.broadcasted_iota(jnp.int32, (1, TM), 1)
    rel = tgt_ref[...] - j * TM
    tmask = col == rel
    n_valid = M - j * TM

    for xn, f_ref, acc, ta in ((xn0, f0_ref, acc0, ta0),
                               (xn1, f1_ref, acc1, ta1),
                               (xn2, f2_ref, acc2, ta2)):
        s = jax.lax.dot_general(xn[...], f_ref[...],
                                dimension_numbers=(((1,), (1,)), ((), ())),
                                preferred_element_type=jnp.float32)
        ta[...] += _fold4(jnp.where(tmask, s, 0.0))
        e = jnp.exp(s)

        @pl.when(j < NT - 1)
        def _full():
            acc[...] += _fold4(e)

        @pl.when(j == NT - 1)
        def _partial():
            acc[...] += _fold4(jnp.where(col < n_valid, e, 0.0))

    @pl.when(j == NT - 1)
    def _epilogue():
        loss = jnp.float32(0.0)
        for k, (xn, acc, ta, te_ref) in enumerate(
                ((xn0, acc0, ta0, t_ref),
                 (xn1, acc1, ta1, tu_ref),
                 (xn2, acc2, ta2, td_ref))):
            lse = jnp.log(jnp.sum(acc[...], axis=1, keepdims=True))
            tgt_logit = jnp.sum(ta[...], axis=1, keepdims=True)
            ce = jnp.sum(lse - tgt_logit) * (1.0 / B)
            xn_plain = xn[...] * TEMP
            tn = _norm_rows(te_ref[...])
            distill = jnp.sum((xn_plain - tn) ** 2) * (1.0 / B)
            w = (1.0 - LAMBDA2) if k == 0 else LAMBDA2
            loss = loss + w * (ce + MU * distill)
        out_ref[...] = jnp.reshape(loss, (1, 1))


def kernel(inputs, inputs_up, inputs_down, inputs_teacher, inputs_up_teacher,
           inputs_down_teacher, targets, epoch, features, features_up,
           features_down):
    del epoch
    tgt2d = targets.astype(jnp.int32).reshape(B, 1)

    full = pl.BlockSpec((B, D), lambda j: (0, 0))
    fspec = pl.BlockSpec((TM, D), lambda j: (j, 0))

    out = pl.pallas_call(
        _loss_kernel,
        grid=(NT,),
        in_specs=[full, full, full, full, full, full,
                  pl.BlockSpec((B, 1), lambda j: (0, 0)),
                  fspec, fspec, fspec],
        out_specs=pl.BlockSpec((1, 1), lambda j: (0, 0)),
        out_shape=jax.ShapeDtypeStruct((1, 1), jnp.float32),
        scratch_shapes=[pltpu.VMEM((B, D), jnp.float32)] * 3
                       + [pltpu.VMEM((B, 128), jnp.float32)] * 6,
        compiler_params=pltpu.CompilerParams(
            dimension_semantics=("arbitrary",)),
    )(inputs, inputs_up, inputs_down, inputs_teacher, inputs_up_teacher,
      inputs_down_teacher, tgt2d, features, features_up, features_down)
    return out[0, 0]
